# baseline (speedup 1.0000x reference)
import functools

import jax
import jax.numpy as jnp
from jax import lax
from jax.experimental import pallas as pl
from jax.experimental.pallas import tpu as pltpu
from jax.experimental.pallas import tpu_sc as plsc

N = 10000
IN_CH = 256
HID = 512
OUT_CH = 256
E_RAW = 160000

N_PAD = 10240
DUMMY = N
NC, NS, L = 2, 16, 16
NW = NC * NS
CB = 128
E_TOT = E_RAW + N
EPT_CHUNKS = -(-E_TOT // (NW * CB))
EPT = EPT_CHUNKS * CB
E_PAD = NW * EPT
RPT = N_PAD // NS
RB = 512

NB0, NB1 = 68, 16
E0 = NS * NB0 * CB

@functools.cache
def _mesh():
    return plsc.VectorSubcoreMesh(core_axis_name="c", subcore_axis_name="s",
                                  num_cores=NC, num_subcores=NS)


def _deg_body(dst_hbm, ones_hbm, zeros_hbm, deg_out, dst_v, ones_v, acc):
    c = lax.axis_index("c")
    s = lax.axis_index("s")
    wid = c * NS + s
    pltpu.sync_copy(dst_hbm.at[wid], dst_v)
    pltpu.sync_copy(ones_hbm, ones_v)
    pltpu.sync_copy(zeros_hbm, acc.at[pl.ds(s * RPT, RPT)])
    plsc.subcore_barrier()

    def body(j, carry):
        pltpu.sync_copy(ones_v, acc.at[dst_v.at[j]], add=True)
        return carry

    lax.fori_loop(0, EPT_CHUNKS, body, 0)
    plsc.subcore_barrier()
    pltpu.sync_copy(acc.at[pl.ds(s * RPT, RPT)], deg_out.at[c, pl.ds(s * RPT, RPT)])


@functools.cache
def _deg_call():
    return pl.kernel(
        _deg_body,
        out_type=jax.ShapeDtypeStruct((NC, N_PAD, 128), jnp.float32),
        mesh=_mesh(),
        scratch_types=[
            pltpu.VMEM((EPT_CHUNKS, CB), jnp.int32),
            pltpu.VMEM((CB, 128), jnp.float32),
            pltpu.VMEM_SHARED((N_PAD, 128), jnp.float32),
        ],
    )


KG = 1


def _agg_body(nch, nb0, nb1, table_hbm, src0_hbm, dst0_hbm, src1_hbm,
              dst1_hbm, zeros_hbm, out_hbm, src_v, dst_v,
              off0, r0, acc, sem0):
    offs = (off0,)
    rows = (r0,)
    sems = (sem0,)
    c = lax.axis_index("c")
    s = lax.axis_index("s")
    nbg = jnp.where(c == 0, nb0 // KG, nb1 // KG)

    @pl.when(c == 0)
    def _():
        pltpu.sync_copy(src0_hbm.at[s], src_v.at[pl.ds(0, nb0)])
        pltpu.sync_copy(dst0_hbm.at[s], dst_v.at[pl.ds(0, nb0)])

    @pl.when(c == 1)
    def _():
        pltpu.sync_copy(src1_hbm.at[s], src_v.at[pl.ds(0, nb1)])
        pltpu.sync_copy(dst1_hbm.at[s], dst_v.at[pl.ds(0, nb1)])

    for ch in range(nch):
        pltpu.sync_copy(zeros_hbm, acc.at[pl.ds(s * RPT, RPT)])
        plsc.subcore_barrier()
        base = jnp.int32(ch * N_PAD)

        def body(g, carry):
            descs = []
            for b in range(KG):
                j = g * KG + b
                for k in range(CB // L):
                    offs[b][pl.ds(k * L, L)] = (
                        src_v[j, pl.ds(k * L, L)] + base)
                descs.append(
                    pltpu.async_copy(table_hbm.at[offs[b]], rows[b], sems[b]))
            for b in range(KG):
                descs[b].wait()
            for b in range(KG):
                pltpu.sync_copy(rows[b], acc.at[dst_v.at[g * KG + b]],
                                add=True)
            return carry

        lax.fori_loop(0, nbg, body, 0)
        plsc.subcore_barrier()
        pltpu.sync_copy(acc.at[pl.ds(s * RPT, RPT)],
                        out_hbm.at[c, ch, pl.ds(s * RPT, RPT)])


@functools.cache
def _make_agg(nch, nb0, nb1):
    nbmax = max(nb0, nb1)
    return pl.kernel(
        functools.partial(_agg_body, nch, nb0, nb1),
        out_type=jax.ShapeDtypeStruct((NC, nch, N_PAD, 128), jnp.float32),
        mesh=_mesh(),
        scratch_types=[
            pltpu.VMEM((nbmax, CB), jnp.int32),
            pltpu.VMEM((nbmax, CB), jnp.int32),
            pltpu.VMEM((CB,), jnp.int32),
            pltpu.VMEM((CB, 128), jnp.float32),
            pltpu.VMEM_SHARED((N_PAD, 128), jnp.float32),
            pltpu.SemaphoreType.DMA,
        ],
    )


def _dinv_of(degp):
    deg = degp[0, :, :1] + degp[1, :, :1]
    return jnp.where(deg > 0, lax.rsqrt(jnp.maximum(deg, 1e-12)), 0.0)


def _mm1_body(x_ref, w_ref, degp_ref, out_ref):
    dinv = _dinv_of(degp_ref[...])
    xw = jnp.dot(x_ref[...], w_ref[...], preferred_element_type=jnp.float32)
    out_ref[0] = xw * dinv


def _mm2_body(p_ref, degp_ref, b1_ref, w2_ref, out_ref):
    dinv = _dinv_of(degp_ref[...])
    p = p_ref[...]
    h = jnp.concatenate([p[0, cc] + p[1, cc] for cc in range(HID // 128)],
                        axis=1)
    h = jnp.maximum(h * dinv + b1_ref[...], 0.0)
    out_ref[0] = jnp.dot(h, w2_ref[...],
                         preferred_element_type=jnp.float32) * dinv


def _fin_body(p_ref, degp_ref, b2_ref, out_ref):
    dinv = _dinv_of(degp_ref[...])
    p = p_ref[...]
    o = jnp.concatenate([p[0, cc] + p[1, cc] for cc in range(OUT_CH // 128)],
                        axis=1)
    out_ref[...] = jnp.maximum(o * dinv + b2_ref[...], 0.0)


_mm1_call = pl.pallas_call(
    _mm1_body,
    grid=(HID // 128, N_PAD // RB),
    in_specs=[
        pl.BlockSpec((RB, IN_CH), lambda ci, ri: (ri, 0)),
        pl.BlockSpec((IN_CH, 128), lambda ci, ri: (0, ci)),
        pl.BlockSpec((2, RB, 128), lambda ci, ri: (0, ri, 0)),
    ],
    out_specs=pl.BlockSpec((1, RB, 128), lambda ci, ri: (ci, ri, 0)),
    out_shape=jax.ShapeDtypeStruct((HID // 128, N_PAD, 128), jnp.float32),
)

_mm2_call = pl.pallas_call(
    _mm2_body,
    grid=(OUT_CH // 128, N_PAD // RB),
    in_specs=[
        pl.BlockSpec((2, HID // 128, RB, 128), lambda ci, ri: (0, 0, ri, 0)),
        pl.BlockSpec((2, RB, 128), lambda ci, ri: (0, ri, 0)),
        pl.BlockSpec((1, HID), lambda ci, ri: (0, 0)),
        pl.BlockSpec((HID, 128), lambda ci, ri: (0, ci)),
    ],
    out_specs=pl.BlockSpec((1, RB, 128), lambda ci, ri: (ci, ri, 0)),
    out_shape=jax.ShapeDtypeStruct((OUT_CH // 128, N_PAD, 128), jnp.float32),
)

_fin_call = pl.pallas_call(
    _fin_body,
    grid=(N_PAD // RB,),
    in_specs=[
        pl.BlockSpec((2, OUT_CH // 128, RB, 128), lambda ri: (0, 0, ri, 0)),
        pl.BlockSpec((2, RB, 128), lambda ri: (0, ri, 0)),
        pl.BlockSpec((1, OUT_CH), lambda ri: (0, 0)),
    ],
    out_specs=pl.BlockSpec((RB, OUT_CH), lambda ri: (ri, 0)),
    out_shape=jax.ShapeDtypeStruct((N_PAD, OUT_CH), jnp.float32),
)


def kernel(x, edge_index, W1, b1, W2, b2):
    loops = jnp.arange(N, dtype=edge_index.dtype)
    pad = jnp.full((E_PAD - E_TOT,), DUMMY, dtype=jnp.int32)
    flat_src = jnp.concatenate(
        [edge_index[0].astype(jnp.int32), loops.astype(jnp.int32), pad])
    flat_dst = jnp.concatenate(
        [edge_index[1].astype(jnp.int32), loops.astype(jnp.int32), pad])
    dstr = flat_dst.reshape(NW, EPT_CHUNKS, CB)
    src0 = flat_src[:E0].reshape(NS, NB0, CB)
    dst0 = flat_dst[:E0].reshape(NS, NB0, CB)
    src1 = flat_src[E0:].reshape(NS, NB1, CB)
    dst1 = flat_dst[E0:].reshape(NS, NB1, CB)

    ones_r = jnp.ones((CB, 128), jnp.float32)
    zerosr = jnp.zeros((RPT, 128), jnp.float32)

    degp = _deg_call()(dstr, ones_r, zerosr)
    x_pad = jnp.pad(x, ((0, N_PAD - N), (0, 0)))
    t1 = _mm1_call(x_pad, W1, degp)
    p1 = _make_agg(HID // 128, NB0, NB1)(
        t1.reshape(HID // 128 * N_PAD, 128),
        src0, dst0, src1, dst1, zerosr)
    t2 = _mm2_call(p1, degp, b1.reshape(1, HID), W2)
    p2 = _make_agg(OUT_CH // 128, NB0, NB1)(
        t2.reshape(OUT_CH // 128 * N_PAD, 128),
        src0, dst0, src1, dst1, zerosr)
    outp = _fin_call(p2, degp, b2.reshape(1, OUT_CH))
    return outp[:N]


# speedup vs baseline: 1.0190x; 1.0190x over previous
import functools

import jax
import jax.numpy as jnp
from jax import lax
from jax.experimental import pallas as pl
from jax.experimental.pallas import tpu as pltpu
from jax.experimental.pallas import tpu_sc as plsc

N = 10000
IN_CH = 256
HID = 512
OUT_CH = 256
E_RAW = 160000

N_PAD = 10240
DUMMY = N
NC, NS, L = 2, 16, 16
NW = NC * NS
CB = 128
E_TOT = E_RAW + N
EPT_CHUNKS = -(-E_TOT // (NW * CB))
EPT = EPT_CHUNKS * CB
E_PAD = NW * EPT
RPT = N_PAD // NS
RB = 512

NB0, NB1 = 66, 18
E0 = NS * NB0 * CB

@functools.cache
def _mesh():
    return plsc.VectorSubcoreMesh(core_axis_name="c", subcore_axis_name="s",
                                  num_cores=NC, num_subcores=NS)


def _deg_body(dst_hbm, ones_hbm, zeros_hbm, deg_out, dst_v, ones_v, acc):
    c = lax.axis_index("c")
    s = lax.axis_index("s")
    wid = c * NS + s
    pltpu.sync_copy(dst_hbm.at[wid], dst_v)
    pltpu.sync_copy(ones_hbm, ones_v)
    pltpu.sync_copy(zeros_hbm, acc.at[pl.ds(s * RPT, RPT)])
    plsc.subcore_barrier()

    def body(j, carry):
        pltpu.sync_copy(ones_v, acc.at[dst_v.at[j]], add=True)
        return carry

    lax.fori_loop(0, EPT_CHUNKS, body, 0)
    plsc.subcore_barrier()
    pltpu.sync_copy(acc.at[pl.ds(s * RPT, RPT)], deg_out.at[c, pl.ds(s * RPT, RPT)])


@functools.cache
def _deg_call():
    return pl.kernel(
        _deg_body,
        out_type=jax.ShapeDtypeStruct((NC, N_PAD, 128), jnp.float32),
        mesh=_mesh(),
        scratch_types=[
            pltpu.VMEM((EPT_CHUNKS, CB), jnp.int32),
            pltpu.VMEM((CB, 128), jnp.float32),
            pltpu.VMEM_SHARED((N_PAD, 128), jnp.float32),
        ],
    )


KG = 1


def _agg_body(nch, nb0, nb1, table_hbm, src0_hbm, dst0_hbm, src1_hbm,
              dst1_hbm, zeros_hbm, out_hbm, src_v, dst_v,
              off0, r0, acc, sem0):
    offs = (off0,)
    rows = (r0,)
    sems = (sem0,)
    c = lax.axis_index("c")
    s = lax.axis_index("s")
    nbg = jnp.where(c == 0, nb0 // KG, nb1 // KG)

    @pl.when(c == 0)
    def _():
        pltpu.sync_copy(src0_hbm.at[s], src_v.at[pl.ds(0, nb0)])
        pltpu.sync_copy(dst0_hbm.at[s], dst_v.at[pl.ds(0, nb0)])

    @pl.when(c == 1)
    def _():
        pltpu.sync_copy(src1_hbm.at[s], src_v.at[pl.ds(0, nb1)])
        pltpu.sync_copy(dst1_hbm.at[s], dst_v.at[pl.ds(0, nb1)])

    for ch in range(nch):
        pltpu.sync_copy(zeros_hbm, acc.at[pl.ds(s * RPT, RPT)])
        plsc.subcore_barrier()
        base = jnp.int32(ch * N_PAD)

        def body(g, carry):
            descs = []
            for b in range(KG):
                j = g * KG + b
                for k in range(CB // L):
                    offs[b][pl.ds(k * L, L)] = (
                        src_v[j, pl.ds(k * L, L)] + base)
                descs.append(
                    pltpu.async_copy(table_hbm.at[offs[b]], rows[b], sems[b]))
            for b in range(KG):
                descs[b].wait()
            for b in range(KG):
                pltpu.sync_copy(rows[b], acc.at[dst_v.at[g * KG + b]],
                                add=True)
            return carry

        lax.fori_loop(0, nbg, body, 0)
        plsc.subcore_barrier()
        pltpu.sync_copy(acc.at[pl.ds(s * RPT, RPT)],
                        out_hbm.at[c, ch, pl.ds(s * RPT, RPT)])


@functools.cache
def _make_agg(nch, nb0, nb1):
    nbmax = max(nb0, nb1)
    return pl.kernel(
        functools.partial(_agg_body, nch, nb0, nb1),
        out_type=jax.ShapeDtypeStruct((NC, nch, N_PAD, 128), jnp.float32),
        mesh=_mesh(),
        scratch_types=[
            pltpu.VMEM((nbmax, CB), jnp.int32),
            pltpu.VMEM((nbmax, CB), jnp.int32),
            pltpu.VMEM((CB,), jnp.int32),
            pltpu.VMEM((CB, 128), jnp.float32),
            pltpu.VMEM_SHARED((N_PAD, 128), jnp.float32),
            pltpu.SemaphoreType.DMA,
        ],
    )


def _dinv_of(degp):
    deg = degp[0, :, :1] + degp[1, :, :1]
    return jnp.where(deg > 0, lax.rsqrt(jnp.maximum(deg, 1e-12)), 0.0)


def _mm1_body(x_ref, w_ref, degp_ref, out_ref):
    dinv = _dinv_of(degp_ref[...])
    xw = jnp.dot(x_ref[...], w_ref[...], preferred_element_type=jnp.float32)
    out_ref[0] = xw * dinv


def _mm2_body(p_ref, degp_ref, b1_ref, w2_ref, out_ref):
    dinv = _dinv_of(degp_ref[...])
    p = p_ref[...]
    h = jnp.concatenate([p[0, cc] + p[1, cc] for cc in range(HID // 128)],
                        axis=1)
    h = jnp.maximum(h * dinv + b1_ref[...], 0.0)
    out_ref[0] = jnp.dot(h, w2_ref[...],
                         preferred_element_type=jnp.float32) * dinv


def _fin_body(p_ref, degp_ref, b2_ref, out_ref):
    dinv = _dinv_of(degp_ref[...])
    p = p_ref[...]
    o = jnp.concatenate([p[0, cc] + p[1, cc] for cc in range(OUT_CH // 128)],
                        axis=1)
    out_ref[...] = jnp.maximum(o * dinv + b2_ref[...], 0.0)


_mm1_call = pl.pallas_call(
    _mm1_body,
    grid=(HID // 128, N_PAD // RB),
    in_specs=[
        pl.BlockSpec((RB, IN_CH), lambda ci, ri: (ri, 0)),
        pl.BlockSpec((IN_CH, 128), lambda ci, ri: (0, ci)),
        pl.BlockSpec((2, RB, 128), lambda ci, ri: (0, ri, 0)),
    ],
    out_specs=pl.BlockSpec((1, RB, 128), lambda ci, ri: (ci, ri, 0)),
    out_shape=jax.ShapeDtypeStruct((HID // 128, N_PAD, 128), jnp.float32),
)

_mm2_call = pl.pallas_call(
    _mm2_body,
    grid=(OUT_CH // 128, N_PAD // RB),
    in_specs=[
        pl.BlockSpec((2, HID // 128, RB, 128), lambda ci, ri: (0, 0, ri, 0)),
        pl.BlockSpec((2, RB, 128), lambda ci, ri: (0, ri, 0)),
        pl.BlockSpec((1, HID), lambda ci, ri: (0, 0)),
        pl.BlockSpec((HID, 128), lambda ci, ri: (0, ci)),
    ],
    out_specs=pl.BlockSpec((1, RB, 128), lambda ci, ri: (ci, ri, 0)),
    out_shape=jax.ShapeDtypeStruct((OUT_CH // 128, N_PAD, 128), jnp.float32),
)

_fin_call = pl.pallas_call(
    _fin_body,
    grid=(N_PAD // RB,),
    in_specs=[
        pl.BlockSpec((2, OUT_CH // 128, RB, 128), lambda ri: (0, 0, ri, 0)),
        pl.BlockSpec((2, RB, 128), lambda ri: (0, ri, 0)),
        pl.BlockSpec((1, OUT_CH), lambda ri: (0, 0)),
    ],
    out_specs=pl.BlockSpec((RB, OUT_CH), lambda ri: (ri, 0)),
    out_shape=jax.ShapeDtypeStruct((N_PAD, OUT_CH), jnp.float32),
)


def kernel(x, edge_index, W1, b1, W2, b2):
    loops = jnp.arange(N, dtype=edge_index.dtype)
    pad = jnp.full((E_PAD - E_TOT,), DUMMY, dtype=jnp.int32)
    flat_src = jnp.concatenate(
        [edge_index[0].astype(jnp.int32), loops.astype(jnp.int32), pad])
    flat_dst = jnp.concatenate(
        [edge_index[1].astype(jnp.int32), loops.astype(jnp.int32), pad])
    dstr = flat_dst.reshape(NW, EPT_CHUNKS, CB)
    src0 = flat_src[:E0].reshape(NS, NB0, CB)
    dst0 = flat_dst[:E0].reshape(NS, NB0, CB)
    src1 = flat_src[E0:].reshape(NS, NB1, CB)
    dst1 = flat_dst[E0:].reshape(NS, NB1, CB)

    ones_r = jnp.ones((CB, 128), jnp.float32)
    zerosr = jnp.zeros((RPT, 128), jnp.float32)

    degp = _deg_call()(dstr, ones_r, zerosr)
    x_pad = jnp.pad(x, ((0, N_PAD - N), (0, 0)))
    t1 = _mm1_call(x_pad, W1, degp)
    p1 = _make_agg(HID // 128, NB0, NB1)(
        t1.reshape(HID // 128 * N_PAD, 128),
        src0, dst0, src1, dst1, zerosr)
    t2 = _mm2_call(p1, degp, b1.reshape(1, HID), W2)
    p2 = _make_agg(OUT_CH // 128, NB0, NB1)(
        t2.reshape(OUT_CH // 128 * N_PAD, 128),
        src0, dst0, src1, dst1, zerosr)
    outp = _fin_call(p2, degp, b2.reshape(1, OUT_CH))
    return outp[:N]


# speedup vs baseline: 1.0401x; 1.0207x over previous
import functools

import jax
import jax.numpy as jnp
from jax import lax
from jax.experimental import pallas as pl
from jax.experimental.pallas import tpu as pltpu
from jax.experimental.pallas import tpu_sc as plsc

N = 10000
IN_CH = 256
HID = 512
OUT_CH = 256
E_RAW = 160000

N_PAD = 10240
DUMMY = N
NC, NS, L = 2, 16, 16
NW = NC * NS
CB = 128
E_TOT = E_RAW + N
EPT_CHUNKS = -(-E_TOT // (NW * CB))
EPT = EPT_CHUNKS * CB
E_PAD = NW * EPT
RPT = N_PAD // NS
RB = 512

NB0, NB1 = 64, 20
E0 = NS * NB0 * CB

@functools.cache
def _mesh():
    return plsc.VectorSubcoreMesh(core_axis_name="c", subcore_axis_name="s",
                                  num_cores=NC, num_subcores=NS)


def _deg_body(dst_hbm, ones_hbm, zeros_hbm, deg_out, dst_v, ones_v, acc):
    c = lax.axis_index("c")
    s = lax.axis_index("s")
    wid = c * NS + s
    pltpu.sync_copy(dst_hbm.at[wid], dst_v)
    pltpu.sync_copy(ones_hbm, ones_v)
    pltpu.sync_copy(zeros_hbm, acc.at[pl.ds(s * RPT, RPT)])
    plsc.subcore_barrier()

    def body(j, carry):
        pltpu.sync_copy(ones_v, acc.at[dst_v.at[j]], add=True)
        return carry

    lax.fori_loop(0, EPT_CHUNKS, body, 0)
    plsc.subcore_barrier()
    pltpu.sync_copy(acc.at[pl.ds(s * RPT, RPT)], deg_out.at[c, pl.ds(s * RPT, RPT)])


@functools.cache
def _deg_call():
    return pl.kernel(
        _deg_body,
        out_type=jax.ShapeDtypeStruct((NC, N_PAD, 128), jnp.float32),
        mesh=_mesh(),
        scratch_types=[
            pltpu.VMEM((EPT_CHUNKS, CB), jnp.int32),
            pltpu.VMEM((CB, 128), jnp.float32),
            pltpu.VMEM_SHARED((N_PAD, 128), jnp.float32),
        ],
    )


KG = 1


def _agg_body(nch, nb0, nb1, table_hbm, src0_hbm, dst0_hbm, src1_hbm,
              dst1_hbm, zeros_hbm, out_hbm, src_v, dst_v,
              off0, r0, acc, sem0):
    offs = (off0,)
    rows = (r0,)
    sems = (sem0,)
    c = lax.axis_index("c")
    s = lax.axis_index("s")
    nbg = jnp.where(c == 0, nb0 // KG, nb1 // KG)

    @pl.when(c == 0)
    def _():
        pltpu.sync_copy(src0_hbm.at[s], src_v.at[pl.ds(0, nb0)])
        pltpu.sync_copy(dst0_hbm.at[s], dst_v.at[pl.ds(0, nb0)])

    @pl.when(c == 1)
    def _():
        pltpu.sync_copy(src1_hbm.at[s], src_v.at[pl.ds(0, nb1)])
        pltpu.sync_copy(dst1_hbm.at[s], dst_v.at[pl.ds(0, nb1)])

    for ch in range(nch):
        pltpu.sync_copy(zeros_hbm, acc.at[pl.ds(s * RPT, RPT)])
        plsc.subcore_barrier()
        base = jnp.int32(ch * N_PAD)

        def body(g, carry):
            descs = []
            for b in range(KG):
                j = g * KG + b
                for k in range(CB // L):
                    offs[b][pl.ds(k * L, L)] = (
                        src_v[j, pl.ds(k * L, L)] + base)
                descs.append(
                    pltpu.async_copy(table_hbm.at[offs[b]], rows[b], sems[b]))
            for b in range(KG):
                descs[b].wait()
            for b in range(KG):
                pltpu.sync_copy(rows[b], acc.at[dst_v.at[g * KG + b]],
                                add=True)
            return carry

        lax.fori_loop(0, nbg, body, 0)
        plsc.subcore_barrier()
        pltpu.sync_copy(acc.at[pl.ds(s * RPT, RPT)],
                        out_hbm.at[c, ch, pl.ds(s * RPT, RPT)])


@functools.cache
def _make_agg(nch, nb0, nb1):
    nbmax = max(nb0, nb1)
    return pl.kernel(
        functools.partial(_agg_body, nch, nb0, nb1),
        out_type=jax.ShapeDtypeStruct((NC, nch, N_PAD, 128), jnp.float32),
        mesh=_mesh(),
        scratch_types=[
            pltpu.VMEM((nbmax, CB), jnp.int32),
            pltpu.VMEM((nbmax, CB), jnp.int32),
            pltpu.VMEM((CB,), jnp.int32),
            pltpu.VMEM((CB, 128), jnp.float32),
            pltpu.VMEM_SHARED((N_PAD, 128), jnp.float32),
            pltpu.SemaphoreType.DMA,
        ],
    )


def _dinv_of(degp):
    deg = degp[0, :, :1] + degp[1, :, :1]
    return jnp.where(deg > 0, lax.rsqrt(jnp.maximum(deg, 1e-12)), 0.0)


def _mm1_body(x_ref, w_ref, degp_ref, out_ref):
    dinv = _dinv_of(degp_ref[...])
    xw = jnp.dot(x_ref[...], w_ref[...], preferred_element_type=jnp.float32)
    out_ref[0] = xw * dinv


def _mm2_body(p_ref, degp_ref, b1_ref, w2_ref, out_ref):
    dinv = _dinv_of(degp_ref[...])
    p = p_ref[...]
    h = jnp.concatenate([p[0, cc] + p[1, cc] for cc in range(HID // 128)],
                        axis=1)
    h = jnp.maximum(h * dinv + b1_ref[...], 0.0)
    out_ref[0] = jnp.dot(h, w2_ref[...],
                         preferred_element_type=jnp.float32) * dinv


def _fin_body(p_ref, degp_ref, b2_ref, out_ref):
    dinv = _dinv_of(degp_ref[...])
    p = p_ref[...]
    o = jnp.concatenate([p[0, cc] + p[1, cc] for cc in range(OUT_CH // 128)],
                        axis=1)
    out_ref[...] = jnp.maximum(o * dinv + b2_ref[...], 0.0)


_mm1_call = pl.pallas_call(
    _mm1_body,
    grid=(HID // 128, N_PAD // RB),
    in_specs=[
        pl.BlockSpec((RB, IN_CH), lambda ci, ri: (ri, 0)),
        pl.BlockSpec((IN_CH, 128), lambda ci, ri: (0, ci)),
        pl.BlockSpec((2, RB, 128), lambda ci, ri: (0, ri, 0)),
    ],
    out_specs=pl.BlockSpec((1, RB, 128), lambda ci, ri: (ci, ri, 0)),
    out_shape=jax.ShapeDtypeStruct((HID // 128, N_PAD, 128), jnp.float32),
)

_mm2_call = pl.pallas_call(
    _mm2_body,
    grid=(OUT_CH // 128, N_PAD // RB),
    in_specs=[
        pl.BlockSpec((2, HID // 128, RB, 128), lambda ci, ri: (0, 0, ri, 0)),
        pl.BlockSpec((2, RB, 128), lambda ci, ri: (0, ri, 0)),
        pl.BlockSpec((1, HID), lambda ci, ri: (0, 0)),
        pl.BlockSpec((HID, 128), lambda ci, ri: (0, ci)),
    ],
    out_specs=pl.BlockSpec((1, RB, 128), lambda ci, ri: (ci, ri, 0)),
    out_shape=jax.ShapeDtypeStruct((OUT_CH // 128, N_PAD, 128), jnp.float32),
)

_fin_call = pl.pallas_call(
    _fin_body,
    grid=(N_PAD // RB,),
    in_specs=[
        pl.BlockSpec((2, OUT_CH // 128, RB, 128), lambda ri: (0, 0, ri, 0)),
        pl.BlockSpec((2, RB, 128), lambda ri: (0, ri, 0)),
        pl.BlockSpec((1, OUT_CH), lambda ri: (0, 0)),
    ],
    out_specs=pl.BlockSpec((RB, OUT_CH), lambda ri: (ri, 0)),
    out_shape=jax.ShapeDtypeStruct((N_PAD, OUT_CH), jnp.float32),
)


def kernel(x, edge_index, W1, b1, W2, b2):
    loops = jnp.arange(N, dtype=edge_index.dtype)
    pad = jnp.full((E_PAD - E_TOT,), DUMMY, dtype=jnp.int32)
    flat_src = jnp.concatenate(
        [edge_index[0].astype(jnp.int32), loops.astype(jnp.int32), pad])
    flat_dst = jnp.concatenate(
        [edge_index[1].astype(jnp.int32), loops.astype(jnp.int32), pad])
    dstr = flat_dst.reshape(NW, EPT_CHUNKS, CB)
    src0 = flat_src[:E0].reshape(NS, NB0, CB)
    dst0 = flat_dst[:E0].reshape(NS, NB0, CB)
    src1 = flat_src[E0:].reshape(NS, NB1, CB)
    dst1 = flat_dst[E0:].reshape(NS, NB1, CB)

    ones_r = jnp.ones((CB, 128), jnp.float32)
    zerosr = jnp.zeros((RPT, 128), jnp.float32)

    degp = _deg_call()(dstr, ones_r, zerosr)
    x_pad = jnp.pad(x, ((0, N_PAD - N), (0, 0)))
    t1 = _mm1_call(x_pad, W1, degp)
    p1 = _make_agg(HID // 128, NB0, NB1)(
        t1.reshape(HID // 128 * N_PAD, 128),
        src0, dst0, src1, dst1, zerosr)
    t2 = _mm2_call(p1, degp, b1.reshape(1, HID), W2)
    p2 = _make_agg(OUT_CH // 128, NB0, NB1)(
        t2.reshape(OUT_CH // 128 * N_PAD, 128),
        src0, dst0, src1, dst1, zerosr)
    outp = _fin_call(p2, degp, b2.reshape(1, OUT_CH))
    return outp[:N]
